# Initial kernel scaffold; baseline (speedup 1.0000x reference)
import jax
import jax.numpy as jnp
from jax.experimental import pallas as pl


def kernel(x, emb_weight):
    raise NotImplementedError("write your pallas kernel here")


# speedup vs baseline: 2.3915x; 2.3915x over previous
import jax
import jax.numpy as jnp
from jax.experimental import pallas as pl


def _scale_copy_block(w_ref, o_ref, *, scale):
    o_ref[...] = w_ref[...] * scale


def kernel(x, emb_weight):
    seq_len = x.shape[1]
    max_seq, dim = emb_weight.shape
    assert seq_len <= max_seq
    scale = dim ** (-0.5)
    block_rows = 512
    grid = (seq_len // block_rows,)
    import functools
    return pl.pallas_call(
        functools.partial(_scale_copy_block, scale=scale),
        grid=grid,
        in_specs=[pl.BlockSpec((block_rows, dim), lambda i: (i, 0))],
        out_specs=pl.BlockSpec((block_rows, dim), lambda i: (i, 0)),
        out_shape=jax.ShapeDtypeStruct((seq_len, dim), emb_weight.dtype),
    )(emb_weight)
